# baseline (speedup 1.0000x reference)
import functools
import math

import jax
import jax.numpy as jnp
from jax import lax
from jax.experimental import pallas as pl
from jax.experimental.pallas import tpu as pltpu
from jax.experimental.pallas import tpu_sc as plsc

NC = 2
NS = 16
NW = NC * NS
B = 128
CC = 32
C2 = 2 * CC


def _mesh():
    return plsc.VectorSubcoreMesh(
        core_axis_name="c", subcore_axis_name="s", num_cores=NC, num_subcores=NS
    )


@functools.lru_cache(maxsize=None)
def _make_degree(NP, NB):
    SR = NP // NS

    @functools.partial(
        pl.kernel,
        out_type=jax.ShapeDtypeStruct((NC, NP), jnp.float32),
        mesh=_mesh(),
        compiler_params=pltpu.CompilerParams(use_tc_tiling_on_sc=False),
        scratch_types=[
            pltpu.VMEM_SHARED((NP,), jnp.float32),
            pltpu.VMEM((NB, B), jnp.int32),
            pltpu.VMEM((NB, B), jnp.int32),
            pltpu.VMEM((NB, B), jnp.float32),
            pltpu.VMEM((B,), jnp.float32),
            pltpu.VMEM((SR,), jnp.float32),
        ],
    )
    def deg(ue, ve, we, dpart, dsh, ubuf, vbuf, wbuf, valbuf, zbuf):
        cid = lax.axis_index("c")
        sid = lax.axis_index("s")
        w = cid * NS + sid
        pltpu.sync_copy(ue.at[w], ubuf)
        pltpu.sync_copy(ve.at[w], vbuf)
        pltpu.sync_copy(we.at[w], wbuf)

        def zfill(i, carry):
            zbuf[pl.ds(i * 16, 16)] = jnp.zeros((16,), jnp.float32)
            return carry

        lax.fori_loop(0, SR // 16, zfill, 0)
        pltpu.sync_copy(zbuf, dsh.at[pl.ds(sid * SR, SR)])
        plsc.subcore_barrier()

        def batch(bi, carry):
            def vfill(j, c2):
                valbuf[pl.ds(j * 16, 16)] = wbuf[bi, pl.ds(j * 16, 16)] * 0.5
                return c2

            lax.fori_loop(0, B // 16, vfill, 0)
            pltpu.sync_copy(valbuf, dsh.at[ubuf.at[bi]], add=True)
            pltpu.sync_copy(valbuf, dsh.at[vbuf.at[bi]], add=True)
            return carry

        lax.fori_loop(0, NB, batch, 0)
        plsc.subcore_barrier()
        pltpu.sync_copy(dsh.at[pl.ds(sid * SR, SR)], dpart.at[cid, pl.ds(sid * SR, SR)])

    return deg


@functools.lru_cache(maxsize=None)
def _make_coef(NP, NB):
    SR = NP // NS

    @functools.partial(
        pl.kernel,
        out_type=[
            jax.ShapeDtypeStruct((NW, NB, B), jnp.float32),
            jax.ShapeDtypeStruct((NW, NB, B), jnp.float32),
        ],
        mesh=_mesh(),
        compiler_params=pltpu.CompilerParams(use_tc_tiling_on_sc=False),
        scratch_types=[
            pltpu.VMEM_SHARED((NP,), jnp.float32),
            pltpu.VMEM((NB, B), jnp.int32),
            pltpu.VMEM((NB, B), jnp.int32),
            pltpu.VMEM((NB, B), jnp.float32),
            pltpu.VMEM((NB, B), jnp.float32),
            pltpu.VMEM((NB, B), jnp.float32),
            pltpu.VMEM((NB, B), jnp.float32),
            pltpu.VMEM((B,), jnp.float32),
            pltpu.VMEM((B,), jnp.float32),
            pltpu.SemaphoreType.DMA,
            pltpu.SemaphoreType.DMA,
        ],
    )
    def coef(dinv, ue, ve, pe, qe, ce, se,
             dsh, ubuf, vbuf, pbuf, qbuf, cbuf, sbuf, gu_b, gv_b, semU, semV):
        cid = lax.axis_index("c")
        sid = lax.axis_index("s")
        w = cid * NS + sid
        stripe = pl.ds(sid * SR, SR)
        pltpu.sync_copy(dinv.at[stripe], dsh.at[stripe])
        pltpu.sync_copy(ue.at[w], ubuf)
        pltpu.sync_copy(ve.at[w], vbuf)
        pltpu.sync_copy(pe.at[w], pbuf)
        pltpu.sync_copy(qe.at[w], qbuf)
        plsc.subcore_barrier()

        def batch(bi, carry):
            gu = pltpu.async_copy(dsh.at[ubuf.at[bi]], gu_b, semU)
            gv = pltpu.async_copy(dsh.at[vbuf.at[bi]], gv_b, semV)
            gu.wait()
            gv.wait()

            def lane(j, c2):
                sl = pl.ds(j * 16, 16)
                g = gu_b[sl] * gv_b[sl]
                cbuf[bi, sl] = pbuf[bi, sl] * g
                sbuf[bi, sl] = qbuf[bi, sl] * g
                return c2

            lax.fori_loop(0, B // 16, lane, 0)
            return carry

        lax.fori_loop(0, NB, batch, 0)
        pltpu.sync_copy(cbuf, ce.at[w])
        pltpu.sync_copy(sbuf, se.at[w])

    return coef


G = 15


@functools.lru_cache(maxsize=None)
def _make_layer(NP, NB2, KH):
    SR = NP // NS
    NG = NB2 // G
    K2 = 2 * KH

    @functools.partial(
        pl.kernel,
        out_type=[
            jax.ShapeDtypeStruct((K2, NP, C2), jnp.float32),
            jax.ShapeDtypeStruct((K2, NP, C2), jnp.float32),
        ],
        mesh=_mesh(),
        compiler_params=pltpu.CompilerParams(use_tc_tiling_on_sc=False),
        scratch_types=[
            pltpu.VMEM_SHARED((NP, C2), jnp.float32),
            pltpu.VMEM((G, B), jnp.int32),
            pltpu.VMEM((G, B), jnp.int32),
            pltpu.VMEM((G, B), jnp.float32),
            pltpu.VMEM((G, B), jnp.float32),
        ]
        + [pltpu.VMEM((B, C2), jnp.float32)] * 6
        + [pltpu.SemaphoreType.DMA] * 6,
    )
    def layer(xt, zrs, ue, ve, ce, se, y1, y2,
              acc, ubuf, vbuf, cbuf, sbuf,
              rU0, rV0, rU1, rV1, rU2, rV2,
              gs0, gs1, gs2, ss0, ss1, ss2):
        cid = lax.axis_index("c")
        sid = lax.axis_index("s")
        stripe = pl.ds(sid * SR, SR)
        sets = ((rU0, rV0, gs0, ss0), (rU1, rV1, gs1, ss1), (rU2, rV2, gs2, ss2))

        def edges_pass(src):
            def issue_gather(bi, si):
                rU, rV, gs, _ = sets[si]
                pltpu.async_copy(src.at[ubuf.at[bi]], rU, gs)
                pltpu.async_copy(src.at[vbuf.at[bi]], rV, gs)

            def wait_gather(si):
                rU, rV, gs, _ = sets[si]
                pltpu.make_async_copy(src.at[ubuf.at[0]], rU, gs).wait()
                pltpu.make_async_copy(src.at[vbuf.at[0]], rV, gs).wait()

            def issue_scatter(bi, si):
                rU, rV, _, ss = sets[si]
                pltpu.async_copy(rV, acc.at[ubuf.at[bi]], ss, add=True)
                pltpu.async_copy(rU, acc.at[vbuf.at[bi]], ss, add=True)

            def drain_scatter(si):
                rU, rV, _, ss = sets[si]
                pltpu.make_async_copy(rV, acc.at[ubuf.at[0]], ss).wait()
                pltpu.make_async_copy(rU, acc.at[vbuf.at[0]], ss).wait()

            def compute(bi, si):
                rU, rV, _, _ = sets[si]

                def group(g, c2):
                    base = g * 16
                    c16 = cbuf[bi, pl.ds(base, 16)]
                    s16 = sbuf[bi, pl.ds(base, 16)]
                    for l in range(16):
                        i = base + l
                        lane = jnp.full((16,), l, jnp.int32)
                        c = jnp.take_along_axis(c16, lane, axis=0)
                        s = jnp.take_along_axis(s16, lane, axis=0)
                        for h in (0, 16):
                            slr = pl.ds(h, 16)
                            sli = pl.ds(CC + h, 16)
                            xr = rV[i, slr]
                            xi = rV[i, sli]
                            rV[i, slr] = c * xr - s * xi
                            rV[i, sli] = s * xr + c * xi
                            ur = rU[i, slr]
                            ui = rU[i, sli]
                            rU[i, slr] = c * ur + s * ui
                            rU[i, sli] = c * ui - s * ur
                    return c2

                lax.fori_loop(0, B // 16, group, 0, unroll=2)

            def grp(gi, carry):
                gsl = pl.ds(gi * G, G)
                pltpu.sync_copy(ue.at[sid, gsl], ubuf)
                pltpu.sync_copy(ve.at[sid, gsl], vbuf)
                pltpu.sync_copy(ce.at[sid, gsl], cbuf)
                pltpu.sync_copy(se.at[sid, gsl], sbuf)
                issue_gather(0, 0)
                issue_gather(1, 1)

                def triple(t, c2):
                    for pp in range(3):
                        bi = t * 3 + pp
                        q = (pp + 2) % 3
                        wait_gather(pp)
                        compute(bi, pp)
                        issue_scatter(bi, pp)
                        if pp == 0:
                            @pl.when(t >= 1)
                            def _():
                                drain_scatter(q)

                            issue_gather(bi + 2, q)
                        else:
                            drain_scatter(q)

                            @pl.when(t <= G // 3 - 2)
                            def _():
                                issue_gather(bi + 2, q)
                    return c2

                lax.fori_loop(0, G // 3, triple, 0)
                drain_scatter((G - 1) % 3)
                return carry

            lax.fori_loop(0, NG, grp, 0)

        for k in range(KH):
            kidx = cid * KH + k
            pltpu.sync_copy(zrs.at[stripe], acc.at[stripe])
            plsc.subcore_barrier()
            edges_pass(xt.at[kidx])
            plsc.subcore_barrier()
            pltpu.sync_copy(acc.at[stripe], y1.at[kidx, stripe])
            pltpu.sync_copy(zrs.at[stripe], acc.at[stripe])
            plsc.subcore_barrier()
            edges_pass(y1.at[kidx])
            plsc.subcore_barrier()
            pltpu.sync_copy(acc.at[stripe], y2.at[kidx, stripe])

    return layer


def _prep_call(dpart, w2d, qs):
    NP = dpart.shape[1]
    NWp, EB = w2d.shape

    def body(q_ref, dpart_ref, w_ref, dinv_ref, pe_ref, qe_ref):
        d = dpart_ref[0:1, :] + dpart_ref[1:2, :]
        dinv_ref[...] = jnp.where(d > 0, lax.rsqrt(d), 0.0)
        th = 2.0 * math.pi * q_ref[0, 0]
        wv = w_ref[...]
        a = 0.5 * wv
        pe_ref[...] = -jnp.cos(th * wv) * a
        qe_ref[...] = -jnp.sin(th * wv) * a

    return pl.pallas_call(
        body,
        out_shape=[
            jax.ShapeDtypeStruct((1, NP), jnp.float32),
            jax.ShapeDtypeStruct((NWp, EB), jnp.float32),
            jax.ShapeDtypeStruct((NWp, EB), jnp.float32),
        ],
        in_specs=[
            pl.BlockSpec(memory_space=pltpu.SMEM),
            pl.BlockSpec((NC, NP), lambda: (0, 0)),
            pl.BlockSpec((NWp, EB), lambda: (0, 0)),
        ],
        out_specs=[
            pl.BlockSpec((1, NP), lambda: (0, 0)),
            pl.BlockSpec((NWp, EB), lambda: (0, 0)),
            pl.BlockSpec((NWp, EB), lambda: (0, 0)),
        ],
    )(qs, dpart, w2d)


def _combine_call(Xr, Xi, y1re, y1im, y2re, y2im, W, b):
    NP, C = Xr.shape
    NF = W.shape[2]
    R = 512
    GR = NP // R

    def body(xr_ref, xi_ref, a_ref, b_ref_, c_ref, d_ref,
             w0_ref, w1_ref, w2_ref, bias_ref, r_ref, i_ref):
        xr = xr_ref[...]
        xi = xi_ref[...]
        y1re_ = a_ref[...]
        y1im_ = b_ref_[...]
        y2re_ = c_ref[...]
        y2im_ = d_ref[...]
        w0 = w0_ref[...]
        w1 = w1_ref[...]
        w2 = w2_ref[...]
        bias = bias_ref[...]
        dot = lambda x, w: jnp.dot(x, w, preferred_element_type=jnp.float32)
        real = dot(-xi, w0) + dot(y1im_, w1) + dot(2.0 * y2im_ + xi, w2) + bias
        imag = dot(xr, w0) + dot(y1re_, w1) + dot(2.0 * y2re_ - xr, w2) + bias
        keep = real >= 0
        r_ref[...] = jnp.where(keep, real, 0.0)
        i_ref[...] = jnp.where(keep, imag, 0.0)

    row = lambda g: (g, 0)
    rep = lambda g: (0, 0)
    return pl.pallas_call(
        body,
        grid=(GR,),
        out_shape=[
            jax.ShapeDtypeStruct((NP, NF), jnp.float32),
            jax.ShapeDtypeStruct((NP, NF), jnp.float32),
        ],
        in_specs=[pl.BlockSpec((R, C), row)] * 6
        + [pl.BlockSpec((C, NF), rep)] * 3
        + [pl.BlockSpec((1, NF), rep)],
        out_specs=[pl.BlockSpec((R, NF), row)] * 2,
    )(Xr, Xi, y1re, y1im, y2re, y2im, W[0], W[1], W[2], b)


def _final_call(r2, i2, y1re, y1im, y2re, y2im, W, b, WcT, bc):
    NP, C = r2.shape
    NF = W.shape[2]
    LB = WcT.shape[1]
    R = 512
    GR = NP // R

    def body(xr_ref, xi_ref, a_ref, b_ref_, c_ref, d_ref,
             w0_ref, w1_ref, w2_ref, bias_ref, wc_ref, bc_ref, o_ref):
        xr = xr_ref[...]
        xi = xi_ref[...]
        y1re_ = a_ref[...]
        y1im_ = b_ref_[...]
        y2re_ = c_ref[...]
        y2im_ = d_ref[...]
        w0 = w0_ref[...]
        w1 = w1_ref[...]
        w2 = w2_ref[...]
        bias = bias_ref[...]
        dot = lambda x, w: jnp.dot(x, w, preferred_element_type=jnp.float32)
        real = dot(-xi, w0) + dot(y1im_, w1) + dot(2.0 * y2im_ + xi, w2) + bias
        imag = dot(xr, w0) + dot(y1re_, w1) + dot(2.0 * y2re_ - xr, w2) + bias
        keep = real >= 0
        r = jnp.where(keep, real, 0.0)
        i = jnp.where(keep, imag, 0.0)
        x = jnp.concatenate([r, i], axis=1)
        logits = dot(x, wc_ref[...]) + bc_ref[...]
        m = jnp.max(logits, axis=-1, keepdims=True)
        lse = jnp.log(jnp.sum(jnp.exp(logits - m), axis=-1, keepdims=True)) + m
        o_ref[...] = logits - lse

    row = lambda g: (g, 0)
    rep = lambda g: (0, 0)
    return pl.pallas_call(
        body,
        grid=(GR,),
        out_shape=jax.ShapeDtypeStruct((NP, LB), jnp.float32),
        in_specs=[pl.BlockSpec((R, C), row)] * 6
        + [pl.BlockSpec((C, NF), rep)] * 3
        + [pl.BlockSpec((1, NF), rep)]
        + [pl.BlockSpec((2 * NF, LB), rep), pl.BlockSpec((1, LB), rep)],
        out_specs=pl.BlockSpec((R, LB), row),
    )(r2, i2, y1re, y1im, y2re, y2im, W[0], W[1], W[2], b, WcT, bc)


def _pack_chunks(r, mi, K):
    return jnp.stack(
        [jnp.concatenate([r[:, k * CC:(k + 1) * CC], mi[:, k * CC:(k + 1) * CC]], axis=1)
         for k in range(K)], axis=0)


def _unpack(y, K):
    re = jnp.concatenate([y[k][:, :CC] for k in range(K)], axis=1)
    im = jnp.concatenate([y[k][:, CC:] for k in range(K)], axis=1)
    return re, im


def _relayout_edges(flat, E, NS_, NB2, fill):
    pad = NS_ * NB2 * B - E
    if fill == "idx":
        padv = jnp.arange(pad, dtype=flat.dtype) % 10000
    else:
        padv = jnp.zeros((pad,), flat.dtype)
    return jnp.concatenate([flat[:E], padv]).reshape(NS_, NB2, B)


def kernel(real, imag, edges, q, edge_weight, W1, b1, W2, b2, Wc, bc):
    N, C = real.shape
    E = edge_weight.shape[0]
    NP = ((N + NS * 16 - 1) // (NS * 16)) * (NS * 16)
    NB32 = (E + NW * B - 1) // (NW * B)
    EP32 = NW * NB32 * B
    NB2 = (E + NS * B - 1) // (NS * B)
    NB2 = ((NB2 + G - 1) // G) * G

    rows = edges[0].astype(jnp.int32)
    cols = edges[1].astype(jnp.int32)
    pad32 = EP32 - E
    pad_idx32 = jnp.arange(pad32, dtype=jnp.int32) % N
    ue32 = jnp.concatenate([rows, pad_idx32]).reshape(NW, NB32, B)
    ve32 = jnp.concatenate([cols, pad_idx32]).reshape(NW, NB32, B)
    wp = jnp.concatenate([edge_weight, jnp.zeros((pad32,), jnp.float32)])
    we32 = wp.reshape(NW, NB32, B)

    dpart = _make_degree(NP, NB32)(ue32, ve32, we32)
    qs = jnp.reshape(jnp.asarray(q, jnp.float32), (1, 1))
    dinv2d, pe2d, qe2d = _prep_call(dpart, wp.reshape(NW, NB32 * B), qs)
    dinv = dinv2d.reshape(NP)
    pe = pe2d.reshape(NW, NB32, B)
    qe = qe2d.reshape(NW, NB32, B)
    ce32, se32 = _make_coef(NP, NB32)(dinv, ue32, ve32, pe, qe)

    pad16 = NS * NB2 * B - E
    pad_idx16 = jnp.arange(pad16, dtype=jnp.int32) % N
    ue = jnp.concatenate([rows, pad_idx16]).reshape(NS, NB2, B)
    ve = jnp.concatenate([cols, pad_idx16]).reshape(NS, NB2, B)
    ce = _relayout_edges(ce32.reshape(-1), E, NS, NB2, "zero")
    se = _relayout_edges(se32.reshape(-1), E, NS, NB2, "zero")

    zrs = jnp.zeros((NP, C2), jnp.float32)
    rpad = jnp.zeros((NP - N, C), jnp.float32)
    Xr = jnp.concatenate([real, rpad], axis=0)
    Xi = jnp.concatenate([imag, rpad], axis=0)

    K1 = C // CC
    xt1 = _pack_chunks(Xr, -Xi, K1)
    y1a, y2a = _make_layer(NP, NB2, K1 // 2)(xt1, zrs, ue, ve, ce, se)
    y1re, y1im = _unpack(list(y1a), K1)
    y2re, y2im = _unpack(list(y2a), K1)
    r2, i2 = _combine_call(Xr, Xi, y1re, y1im, y2re, y2im, W1, b1)

    NF = W2.shape[2]
    K2 = NF // CC
    xt2 = _pack_chunks(r2, -i2, K2)
    z1a, z2a = _make_layer(NP, NB2, K2 // 2)(xt2, zrs, ue, ve, ce, se)
    z1re, z1im = _unpack(list(z1a), K2)
    z2re, z2im = _unpack(list(z2a), K2)

    out = _final_call(r2, i2, z1re, z1im, z2re, z2im,
                      W2, b2, Wc.T, bc.reshape(1, -1))
    return out[:N]


# speedup vs baseline: 1.0769x; 1.0769x over previous
import functools
import math

import jax
import jax.numpy as jnp
from jax import lax
from jax.experimental import pallas as pl
from jax.experimental.pallas import tpu as pltpu
from jax.experimental.pallas import tpu_sc as plsc

NC = 2
NS = 16
NW = NC * NS
B = 128
CC = 32
C2 = 2 * CC


def _mesh():
    return plsc.VectorSubcoreMesh(
        core_axis_name="c", subcore_axis_name="s", num_cores=NC, num_subcores=NS
    )


@functools.lru_cache(maxsize=None)
def _make_degree(NP, NB):
    SR = NP // NS

    @functools.partial(
        pl.kernel,
        out_type=jax.ShapeDtypeStruct((NC, NP), jnp.float32),
        mesh=_mesh(),
        compiler_params=pltpu.CompilerParams(use_tc_tiling_on_sc=False),
        scratch_types=[
            pltpu.VMEM_SHARED((NP,), jnp.float32),
            pltpu.VMEM((NB, B), jnp.int32),
            pltpu.VMEM((NB, B), jnp.int32),
            pltpu.VMEM((NB, B), jnp.float32),
            pltpu.VMEM((B,), jnp.float32),
            pltpu.VMEM((SR,), jnp.float32),
        ],
    )
    def deg(ue, ve, we, dpart, dsh, ubuf, vbuf, wbuf, valbuf, zbuf):
        cid = lax.axis_index("c")
        sid = lax.axis_index("s")
        w = cid * NS + sid
        pltpu.sync_copy(ue.at[w], ubuf)
        pltpu.sync_copy(ve.at[w], vbuf)
        pltpu.sync_copy(we.at[w], wbuf)

        def zfill(i, carry):
            zbuf[pl.ds(i * 16, 16)] = jnp.zeros((16,), jnp.float32)
            return carry

        lax.fori_loop(0, SR // 16, zfill, 0)
        pltpu.sync_copy(zbuf, dsh.at[pl.ds(sid * SR, SR)])
        plsc.subcore_barrier()

        def batch(bi, carry):
            def vfill(j, c2):
                valbuf[pl.ds(j * 16, 16)] = wbuf[bi, pl.ds(j * 16, 16)] * 0.5
                return c2

            lax.fori_loop(0, B // 16, vfill, 0)
            pltpu.sync_copy(valbuf, dsh.at[ubuf.at[bi]], add=True)
            pltpu.sync_copy(valbuf, dsh.at[vbuf.at[bi]], add=True)
            return carry

        lax.fori_loop(0, NB, batch, 0)
        plsc.subcore_barrier()
        pltpu.sync_copy(dsh.at[pl.ds(sid * SR, SR)], dpart.at[cid, pl.ds(sid * SR, SR)])

    return deg


@functools.lru_cache(maxsize=None)
def _make_coef(NP, NB):
    SR = NP // NS

    @functools.partial(
        pl.kernel,
        out_type=[
            jax.ShapeDtypeStruct((NW, NB, B), jnp.float32),
            jax.ShapeDtypeStruct((NW, NB, B), jnp.float32),
        ],
        mesh=_mesh(),
        compiler_params=pltpu.CompilerParams(use_tc_tiling_on_sc=False),
        scratch_types=[
            pltpu.VMEM_SHARED((NP,), jnp.float32),
            pltpu.VMEM((NB, B), jnp.int32),
            pltpu.VMEM((NB, B), jnp.int32),
            pltpu.VMEM((NB, B), jnp.float32),
            pltpu.VMEM((NB, B), jnp.float32),
            pltpu.VMEM((NB, B), jnp.float32),
            pltpu.VMEM((NB, B), jnp.float32),
            pltpu.VMEM((B,), jnp.float32),
            pltpu.VMEM((B,), jnp.float32),
            pltpu.SemaphoreType.DMA,
            pltpu.SemaphoreType.DMA,
        ],
    )
    def coef(dinv, ue, ve, pe, qe, ce, se,
             dsh, ubuf, vbuf, pbuf, qbuf, cbuf, sbuf, gu_b, gv_b, semU, semV):
        cid = lax.axis_index("c")
        sid = lax.axis_index("s")
        w = cid * NS + sid
        stripe = pl.ds(sid * SR, SR)
        pltpu.sync_copy(dinv.at[stripe], dsh.at[stripe])
        pltpu.sync_copy(ue.at[w], ubuf)
        pltpu.sync_copy(ve.at[w], vbuf)
        pltpu.sync_copy(pe.at[w], pbuf)
        pltpu.sync_copy(qe.at[w], qbuf)
        plsc.subcore_barrier()

        def batch(bi, carry):
            gu = pltpu.async_copy(dsh.at[ubuf.at[bi]], gu_b, semU)
            gv = pltpu.async_copy(dsh.at[vbuf.at[bi]], gv_b, semV)
            gu.wait()
            gv.wait()

            def lane(j, c2):
                sl = pl.ds(j * 16, 16)
                g = gu_b[sl] * gv_b[sl]
                cbuf[bi, sl] = pbuf[bi, sl] * g
                sbuf[bi, sl] = qbuf[bi, sl] * g
                return c2

            lax.fori_loop(0, B // 16, lane, 0)
            return carry

        lax.fori_loop(0, NB, batch, 0)
        pltpu.sync_copy(cbuf, ce.at[w])
        pltpu.sync_copy(sbuf, se.at[w])

    return coef


G = 33


@functools.lru_cache(maxsize=None)
def _make_layer(NP, NB2, KH):
    SR = NP // NS
    NG = NB2 // G
    K2 = 2 * KH

    @functools.partial(
        pl.kernel,
        out_type=[
            jax.ShapeDtypeStruct((K2, NP, C2), jnp.float32),
            jax.ShapeDtypeStruct((K2, NP, C2), jnp.float32),
        ],
        mesh=_mesh(),
        compiler_params=pltpu.CompilerParams(use_tc_tiling_on_sc=False),
        scratch_types=[
            pltpu.VMEM_SHARED((NP, C2), jnp.float32),
            pltpu.VMEM((G, B), jnp.int32),
            pltpu.VMEM((G, B), jnp.int32),
            pltpu.VMEM((G, B), jnp.float32),
            pltpu.VMEM((G, B), jnp.float32),
        ]
        + [pltpu.VMEM((B, C2), jnp.float32)] * 6
        + [pltpu.SemaphoreType.DMA] * 6,
    )
    def layer(xt, zrs, ue, ve, ce, se, y1, y2,
              acc, ubuf, vbuf, cbuf, sbuf,
              rU0, rV0, rU1, rV1, rU2, rV2,
              gs0, gs1, gs2, ss0, ss1, ss2):
        cid = lax.axis_index("c")
        sid = lax.axis_index("s")
        stripe = pl.ds(sid * SR, SR)
        sets = ((rU0, rV0, gs0, ss0), (rU1, rV1, gs1, ss1), (rU2, rV2, gs2, ss2))

        def edges_pass(src):
            def issue_gather(bi, si):
                rU, rV, gs, _ = sets[si]
                pltpu.async_copy(src.at[ubuf.at[bi]], rU, gs)
                pltpu.async_copy(src.at[vbuf.at[bi]], rV, gs)

            def wait_gather(si):
                rU, rV, gs, _ = sets[si]
                pltpu.make_async_copy(src.at[ubuf.at[0]], rU, gs).wait()
                pltpu.make_async_copy(src.at[vbuf.at[0]], rV, gs).wait()

            def issue_scatter(bi, si):
                rU, rV, _, ss = sets[si]
                pltpu.async_copy(rV, acc.at[ubuf.at[bi]], ss, add=True)
                pltpu.async_copy(rU, acc.at[vbuf.at[bi]], ss, add=True)

            def drain_scatter(si):
                rU, rV, _, ss = sets[si]
                pltpu.make_async_copy(rV, acc.at[ubuf.at[0]], ss).wait()
                pltpu.make_async_copy(rU, acc.at[vbuf.at[0]], ss).wait()

            def compute(bi, si):
                rU, rV, _, _ = sets[si]

                def group(g, c2):
                    base = g * 16
                    c16 = cbuf[bi, pl.ds(base, 16)]
                    s16 = sbuf[bi, pl.ds(base, 16)]
                    for l in range(16):
                        i = base + l
                        lane = jnp.full((16,), l, jnp.int32)
                        c = jnp.take_along_axis(c16, lane, axis=0)
                        s = jnp.take_along_axis(s16, lane, axis=0)
                        for h in (0, 16):
                            slr = pl.ds(h, 16)
                            sli = pl.ds(CC + h, 16)
                            xr = rV[i, slr]
                            xi = rV[i, sli]
                            rV[i, slr] = c * xr - s * xi
                            rV[i, sli] = s * xr + c * xi
                            ur = rU[i, slr]
                            ui = rU[i, sli]
                            rU[i, slr] = c * ur + s * ui
                            rU[i, sli] = c * ui - s * ur
                    return c2

                lax.fori_loop(0, B // 16, group, 0)

            def grp(gi, carry):
                gsl = pl.ds(gi * G, G)
                pltpu.sync_copy(ue.at[sid, gsl], ubuf)
                pltpu.sync_copy(ve.at[sid, gsl], vbuf)
                pltpu.sync_copy(ce.at[sid, gsl], cbuf)
                pltpu.sync_copy(se.at[sid, gsl], sbuf)
                issue_gather(0, 0)
                issue_gather(1, 1)

                def triple(t, c2):
                    for pp in range(3):
                        bi = t * 3 + pp
                        q = (pp + 2) % 3
                        wait_gather(pp)
                        compute(bi, pp)
                        issue_scatter(bi, pp)
                        if pp == 0:
                            @pl.when(t >= 1)
                            def _():
                                drain_scatter(q)

                            issue_gather(bi + 2, q)
                        else:
                            drain_scatter(q)

                            @pl.when(t <= G // 3 - 2)
                            def _():
                                issue_gather(bi + 2, q)
                    return c2

                lax.fori_loop(0, G // 3, triple, 0)
                drain_scatter((G - 1) % 3)
                return carry

            lax.fori_loop(0, NG, grp, 0)

        for k in range(KH):
            kidx = cid * KH + k
            pltpu.sync_copy(zrs.at[stripe], acc.at[stripe])
            plsc.subcore_barrier()
            edges_pass(xt.at[kidx])
            plsc.subcore_barrier()
            pltpu.sync_copy(acc.at[stripe], y1.at[kidx, stripe])
            pltpu.sync_copy(zrs.at[stripe], acc.at[stripe])
            plsc.subcore_barrier()
            edges_pass(y1.at[kidx])
            plsc.subcore_barrier()
            pltpu.sync_copy(acc.at[stripe], y2.at[kidx, stripe])

    return layer


def _prep_call(dpart, w2d, qs):
    NP = dpart.shape[1]
    NWp, EB = w2d.shape

    def body(q_ref, dpart_ref, w_ref, dinv_ref, pe_ref, qe_ref):
        d = dpart_ref[0:1, :] + dpart_ref[1:2, :]
        dinv_ref[...] = jnp.where(d > 0, lax.rsqrt(d), 0.0)
        th = 2.0 * math.pi * q_ref[0, 0]
        wv = w_ref[...]
        a = 0.5 * wv
        pe_ref[...] = -jnp.cos(th * wv) * a
        qe_ref[...] = -jnp.sin(th * wv) * a

    return pl.pallas_call(
        body,
        out_shape=[
            jax.ShapeDtypeStruct((1, NP), jnp.float32),
            jax.ShapeDtypeStruct((NWp, EB), jnp.float32),
            jax.ShapeDtypeStruct((NWp, EB), jnp.float32),
        ],
        in_specs=[
            pl.BlockSpec(memory_space=pltpu.SMEM),
            pl.BlockSpec((NC, NP), lambda: (0, 0)),
            pl.BlockSpec((NWp, EB), lambda: (0, 0)),
        ],
        out_specs=[
            pl.BlockSpec((1, NP), lambda: (0, 0)),
            pl.BlockSpec((NWp, EB), lambda: (0, 0)),
            pl.BlockSpec((NWp, EB), lambda: (0, 0)),
        ],
    )(qs, dpart, w2d)


def _combine_call(Xr, Xi, y1re, y1im, y2re, y2im, W, b):
    NP, C = Xr.shape
    NF = W.shape[2]
    R = 512
    GR = NP // R

    def body(xr_ref, xi_ref, a_ref, b_ref_, c_ref, d_ref,
             w0_ref, w1_ref, w2_ref, bias_ref, r_ref, i_ref):
        xr = xr_ref[...]
        xi = xi_ref[...]
        y1re_ = a_ref[...]
        y1im_ = b_ref_[...]
        y2re_ = c_ref[...]
        y2im_ = d_ref[...]
        w0 = w0_ref[...]
        w1 = w1_ref[...]
        w2 = w2_ref[...]
        bias = bias_ref[...]
        dot = lambda x, w: jnp.dot(x, w, preferred_element_type=jnp.float32)
        real = dot(-xi, w0) + dot(y1im_, w1) + dot(2.0 * y2im_ + xi, w2) + bias
        imag = dot(xr, w0) + dot(y1re_, w1) + dot(2.0 * y2re_ - xr, w2) + bias
        keep = real >= 0
        r_ref[...] = jnp.where(keep, real, 0.0)
        i_ref[...] = jnp.where(keep, imag, 0.0)

    row = lambda g: (g, 0)
    rep = lambda g: (0, 0)
    return pl.pallas_call(
        body,
        grid=(GR,),
        out_shape=[
            jax.ShapeDtypeStruct((NP, NF), jnp.float32),
            jax.ShapeDtypeStruct((NP, NF), jnp.float32),
        ],
        in_specs=[pl.BlockSpec((R, C), row)] * 6
        + [pl.BlockSpec((C, NF), rep)] * 3
        + [pl.BlockSpec((1, NF), rep)],
        out_specs=[pl.BlockSpec((R, NF), row)] * 2,
    )(Xr, Xi, y1re, y1im, y2re, y2im, W[0], W[1], W[2], b)


def _final_call(r2, i2, y1re, y1im, y2re, y2im, W, b, WcT, bc):
    NP, C = r2.shape
    NF = W.shape[2]
    LB = WcT.shape[1]
    R = 512
    GR = NP // R

    def body(xr_ref, xi_ref, a_ref, b_ref_, c_ref, d_ref,
             w0_ref, w1_ref, w2_ref, bias_ref, wc_ref, bc_ref, o_ref):
        xr = xr_ref[...]
        xi = xi_ref[...]
        y1re_ = a_ref[...]
        y1im_ = b_ref_[...]
        y2re_ = c_ref[...]
        y2im_ = d_ref[...]
        w0 = w0_ref[...]
        w1 = w1_ref[...]
        w2 = w2_ref[...]
        bias = bias_ref[...]
        dot = lambda x, w: jnp.dot(x, w, preferred_element_type=jnp.float32)
        real = dot(-xi, w0) + dot(y1im_, w1) + dot(2.0 * y2im_ + xi, w2) + bias
        imag = dot(xr, w0) + dot(y1re_, w1) + dot(2.0 * y2re_ - xr, w2) + bias
        keep = real >= 0
        r = jnp.where(keep, real, 0.0)
        i = jnp.where(keep, imag, 0.0)
        x = jnp.concatenate([r, i], axis=1)
        logits = dot(x, wc_ref[...]) + bc_ref[...]
        m = jnp.max(logits, axis=-1, keepdims=True)
        lse = jnp.log(jnp.sum(jnp.exp(logits - m), axis=-1, keepdims=True)) + m
        o_ref[...] = logits - lse

    row = lambda g: (g, 0)
    rep = lambda g: (0, 0)
    return pl.pallas_call(
        body,
        grid=(GR,),
        out_shape=jax.ShapeDtypeStruct((NP, LB), jnp.float32),
        in_specs=[pl.BlockSpec((R, C), row)] * 6
        + [pl.BlockSpec((C, NF), rep)] * 3
        + [pl.BlockSpec((1, NF), rep)]
        + [pl.BlockSpec((2 * NF, LB), rep), pl.BlockSpec((1, LB), rep)],
        out_specs=pl.BlockSpec((R, LB), row),
    )(r2, i2, y1re, y1im, y2re, y2im, W[0], W[1], W[2], b, WcT, bc)


def _pack_chunks(r, mi, K):
    return jnp.stack(
        [jnp.concatenate([r[:, k * CC:(k + 1) * CC], mi[:, k * CC:(k + 1) * CC]], axis=1)
         for k in range(K)], axis=0)


def _unpack(y, K):
    re = jnp.concatenate([y[k][:, :CC] for k in range(K)], axis=1)
    im = jnp.concatenate([y[k][:, CC:] for k in range(K)], axis=1)
    return re, im


def _relayout_edges(flat, E, NS_, NB2, fill):
    pad = NS_ * NB2 * B - E
    if fill == "idx":
        padv = jnp.arange(pad, dtype=flat.dtype) % 10000
    else:
        padv = jnp.zeros((pad,), flat.dtype)
    return jnp.concatenate([flat[:E], padv]).reshape(NS_, NB2, B)


def kernel(real, imag, edges, q, edge_weight, W1, b1, W2, b2, Wc, bc):
    N, C = real.shape
    E = edge_weight.shape[0]
    NP = ((N + NS * 16 - 1) // (NS * 16)) * (NS * 16)
    NB32 = (E + NW * B - 1) // (NW * B)
    EP32 = NW * NB32 * B
    NB2 = (E + NS * B - 1) // (NS * B)
    NB2 = ((NB2 + G - 1) // G) * G

    rows = edges[0].astype(jnp.int32)
    cols = edges[1].astype(jnp.int32)
    pad32 = EP32 - E
    pad_idx32 = jnp.arange(pad32, dtype=jnp.int32) % N
    ue32 = jnp.concatenate([rows, pad_idx32]).reshape(NW, NB32, B)
    ve32 = jnp.concatenate([cols, pad_idx32]).reshape(NW, NB32, B)
    wp = jnp.concatenate([edge_weight, jnp.zeros((pad32,), jnp.float32)])
    we32 = wp.reshape(NW, NB32, B)

    dpart = _make_degree(NP, NB32)(ue32, ve32, we32)
    qs = jnp.reshape(jnp.asarray(q, jnp.float32), (1, 1))
    dinv2d, pe2d, qe2d = _prep_call(dpart, wp.reshape(NW, NB32 * B), qs)
    dinv = dinv2d.reshape(NP)
    pe = pe2d.reshape(NW, NB32, B)
    qe = qe2d.reshape(NW, NB32, B)
    ce32, se32 = _make_coef(NP, NB32)(dinv, ue32, ve32, pe, qe)

    pad16 = NS * NB2 * B - E
    pad_idx16 = jnp.arange(pad16, dtype=jnp.int32) % N
    ue = jnp.concatenate([rows, pad_idx16]).reshape(NS, NB2, B)
    ve = jnp.concatenate([cols, pad_idx16]).reshape(NS, NB2, B)
    ce = _relayout_edges(ce32.reshape(-1), E, NS, NB2, "zero")
    se = _relayout_edges(se32.reshape(-1), E, NS, NB2, "zero")

    zrs = jnp.zeros((NP, C2), jnp.float32)
    rpad = jnp.zeros((NP - N, C), jnp.float32)
    Xr = jnp.concatenate([real, rpad], axis=0)
    Xi = jnp.concatenate([imag, rpad], axis=0)

    K1 = C // CC
    xt1 = _pack_chunks(Xr, -Xi, K1)
    y1a, y2a = _make_layer(NP, NB2, K1 // 2)(xt1, zrs, ue, ve, ce, se)
    y1re, y1im = _unpack(list(y1a), K1)
    y2re, y2im = _unpack(list(y2a), K1)
    r2, i2 = _combine_call(Xr, Xi, y1re, y1im, y2re, y2im, W1, b1)

    NF = W2.shape[2]
    K2 = NF // CC
    xt2 = _pack_chunks(r2, -i2, K2)
    z1a, z2a = _make_layer(NP, NB2, K2 // 2)(xt2, zrs, ue, ve, ce, se)
    z1re, z1im = _unpack(list(z1a), K2)
    z2re, z2im = _unpack(list(z2a), K2)

    out = _final_call(r2, i2, z1re, z1im, z2re, z2im,
                      W2, b2, Wc.T, bc.reshape(1, -1))
    return out[:N]
